# baseline (speedup 1.0000x reference)
import functools

import jax
import jax.numpy as jnp
from jax import lax
from jax.experimental import pallas as pl
from jax.experimental.pallas import tpu as pltpu
from jax.experimental.pallas import tpu_sc as plsc

N = 10000
E = 320000
DIM = 128
C = 10
M = 3

NC = 2
NS = 16
NW = NC * NS
CHUNK = 64
NB = 4
CH_PER_TILE = 160
EPT = CH_PER_TILE * CHUNK
E_PAD = EPT * NW
ROWS_PER_TILE = 640
N_ACC = NS * ROWS_PER_TILE
DCHUNK = 128
DCH = EPT // DCHUNK


def _deg_body(dst_hbm, out_hbm, acc, idxd, vals, ssem):
    cid = lax.axis_index("c")
    sid = lax.axis_index("s")
    wid = sid * NC + cid

    zero16 = jnp.zeros((16,), jnp.float32)

    def _zrow(i, carry):
        vals[i] = zero16
        return carry

    lax.fori_loop(0, DCHUNK, _zrow, 0)
    base_row = sid * ROWS_PER_TILE
    for t in range(ROWS_PER_TILE // DCHUNK):
        pltpu.sync_copy(vals, acc.at[pl.ds(base_row + t * DCHUNK, DCHUNK)])

    one16 = jnp.ones((16,), jnp.float32)

    def _orow(i, carry):
        vals[i] = one16
        return carry

    lax.fori_loop(0, DCHUNK, _orow, 0)
    plsc.subcore_barrier()
    ebase = wid * EPT

    def _chunk(c, carry):
        off = ebase + c * DCHUNK
        pltpu.sync_copy(dst_hbm.at[pl.ds(off, DCHUNK)], idxd)
        pltpu.sync_copy(vals, acc.at[idxd], add=True)
        return carry

    lax.fori_loop(0, DCH, _chunk, 0)
    plsc.subcore_barrier()
    pltpu.sync_copy(acc.at[pl.ds(base_row, ROWS_PER_TILE)],
                    out_hbm.at[cid, pl.ds(base_row, ROWS_PER_TILE)])


@functools.cache
def _deg_call():
    return pl.kernel(
        _deg_body,
        out_type=jax.ShapeDtypeStruct((NC, N_ACC, 16), jnp.float32),
        mesh=plsc.VectorSubcoreMesh(core_axis_name="c", subcore_axis_name="s"),
        scratch_types=[
            pltpu.VMEM_SHARED((N_ACC, 16), jnp.float32),
            pltpu.VMEM((DCHUNK,), jnp.int32),
            pltpu.VMEM((DCHUNK, 16), jnp.float32),
            pltpu.SemaphoreType.DMA,
        ],
    )


def _prop_body(g_hbm, src_hbm, dst_hbm, out_hbm, acc,
               is0, is1, is2, is3, id0, id1, id2, id3,
               r0, r1, r2, r3, ie0, ie1, ie2, ie3,
               gs0, gs1, gs2, gs3, ss0, ss1, ss2, ss3):
    isrc = [is0, is1, is2, is3]
    idst = [id0, id1, id2, id3]
    rows = [r0, r1, r2, r3]
    isem = [ie0, ie1, ie2, ie3]
    gsem = [gs0, gs1, gs2, gs3]
    ssem = [ss0, ss1, ss2, ss3]
    cid = lax.axis_index("c")
    sid = lax.axis_index("s")
    wid = sid * NC + cid

    zero16 = jnp.zeros((16,), jnp.float32)

    def _zrow(i, carry):
        r0[i // 8, pl.ds((i % 8) * 16, 16)] = zero16
        return carry

    lax.fori_loop(0, CHUNK * 8, _zrow, 0)
    base_row = sid * ROWS_PER_TILE
    for t in range(ROWS_PER_TILE // CHUNK):
        pltpu.sync_copy(r0, acc.at[pl.ds(base_row + t * CHUNK, CHUNK)])

    def _i_start(c, b):
        pltpu.async_copy(src_hbm.at[wid, c], isrc[b], isem[b])
        pltpu.async_copy(dst_hbm.at[wid, c], idst[b], isem[b])

    def _i_wait(c, b):
        pltpu.make_async_copy(src_hbm.at[wid, c], isrc[b], isem[b]).wait()
        pltpu.make_async_copy(dst_hbm.at[wid, c], idst[b], isem[b]).wait()

    def _g_start(b):
        pltpu.async_copy(g_hbm.at[isrc[b]], rows[b], gsem[b])

    def _g_wait(b):
        pltpu.make_async_copy(g_hbm.at[isrc[b]], rows[b], gsem[b]).wait()

    def _s_start(b):
        pltpu.async_copy(rows[b], acc.at[idst[b]], ssem[b], add=True)

    def _s_wait(b):
        pltpu.make_async_copy(rows[b], acc.at[idst[b]], ssem[b]).wait()

    for c in range(3):
        _i_start(c, c)
    for c in range(2):
        _i_wait(c, c)
        _g_start(c)
    plsc.subcore_barrier()

    def _step(i, k, sw_ok, is_ok, gs_ok):
        b_prev = (k + 3) % NB
        b_g = (k + 2) % NB
        if sw_ok:
            _s_wait(b_prev)
        if is_ok:
            _i_start(i + 3, b_prev)
        if gs_ok:
            _i_wait(i + 2, b_g)
            _g_start(b_g)
        _g_wait(k)
        _s_start(k)

    for k in range(NB):
        _step(k, k, k >= 1, True, True)

    def _group(g, carry):
        i0 = g * NB
        for k in range(NB):
            _step(i0 + k, k, True, True, True)
        return carry

    lax.fori_loop(1, CH_PER_TILE // NB - 1, _group, 0)

    ilast = CH_PER_TILE - NB
    for k in range(NB):
        i = ilast + k
        _step(i, k, True, i + 3 < CH_PER_TILE, i + 2 < CH_PER_TILE)
    _s_wait((CH_PER_TILE - 1) % NB)

    plsc.subcore_barrier()
    pltpu.sync_copy(acc.at[pl.ds(base_row, ROWS_PER_TILE)],
                    out_hbm.at[cid, pl.ds(base_row, ROWS_PER_TILE)])


@functools.cache
def _prop_call():
    return pl.kernel(
        _prop_body,
        out_type=jax.ShapeDtypeStruct((NC, N_ACC, DIM), jnp.float32),
        mesh=plsc.VectorSubcoreMesh(core_axis_name="c", subcore_axis_name="s"),
        scratch_types=(
            [pltpu.VMEM_SHARED((N_ACC, DIM), jnp.float32)]
            + [pltpu.VMEM((CHUNK,), jnp.int32)] * (2 * NB)
            + [pltpu.VMEM((CHUNK, DIM), jnp.float32)] * NB
            + [pltpu.SemaphoreType.DMA] * (3 * NB)
        ),
    )


_BLK = 1000
_GRID = N // _BLK


def _attention_combine(h_list, att):
    ss = [jnp.dot(jnp.tanh(h), att) for h in h_list]
    mx = jnp.maximum(jnp.maximum(ss[0], ss[1]), ss[2])
    es = [jnp.exp(s - mx) for s in ss]
    z = es[0] + es[1] + es[2]
    return (es[0] * h_list[0] + es[1] * h_list[1] + es[2] * h_list[2]) / z


def _dense1_body(x_ref, w_ref, a_ref, d0_ref, d1_ref, g_ref, dinv_ref):
    dinv = lax.rsqrt(d0_ref[...] + d1_ref[...] + 1.0)
    x = x_ref[...]
    hs = [jnp.dot(x, w_ref[m]) for m in range(M)]
    hagg = _attention_combine(hs, a_ref[...])
    dinv_ref[...] = dinv
    g_ref[...] = hagg * dinv


_dense1 = pl.pallas_call(
    _dense1_body,
    grid=(_GRID,),
    in_specs=[
        pl.BlockSpec((_BLK, DIM), lambda i: (i, 0)),
        pl.BlockSpec((M, DIM, DIM), lambda i: (0, 0, 0)),
        pl.BlockSpec((DIM, 1), lambda i: (0, 0)),
        pl.BlockSpec((_BLK, 1), lambda i: (i, 0)),
        pl.BlockSpec((_BLK, 1), lambda i: (i, 0)),
    ],
    out_specs=[
        pl.BlockSpec((_BLK, DIM), lambda i: (i, 0)),
        pl.BlockSpec((_BLK, 1), lambda i: (i, 0)),
    ],
    out_shape=[
        jax.ShapeDtypeStruct((N, DIM), jnp.float32),
        jax.ShapeDtypeStruct((N, 1), jnp.float32),
    ],
)


def _dense2_body(s0_ref, s1_ref, g1_ref, dinv_ref, w_ref, a_ref, g2_ref):
    dinv = dinv_ref[...]
    h = jnp.maximum(dinv * (s0_ref[...] + s1_ref[...] + g1_ref[...]), 0.0)
    hs = [jnp.dot(h, w_ref[m]) for m in range(M)]
    hagg = _attention_combine(hs, a_ref[...])
    g2_ref[...] = hagg * dinv


_dense2 = pl.pallas_call(
    _dense2_body,
    grid=(_GRID,),
    in_specs=[
        pl.BlockSpec((_BLK, DIM), lambda i: (i, 0)),
        pl.BlockSpec((_BLK, DIM), lambda i: (i, 0)),
        pl.BlockSpec((_BLK, DIM), lambda i: (i, 0)),
        pl.BlockSpec((_BLK, 1), lambda i: (i, 0)),
        pl.BlockSpec((M, DIM, DIM), lambda i: (0, 0, 0)),
        pl.BlockSpec((DIM, 1), lambda i: (0, 0)),
    ],
    out_specs=pl.BlockSpec((_BLK, DIM), lambda i: (i, 0)),
    out_shape=jax.ShapeDtypeStruct((N, DIM), jnp.float32),
)


def _head_body(t0_ref, t1_ref, g2_ref, dinv_ref, wc_ref, bc_ref, ac_ref, o_ref):
    o = dinv_ref[...] * (t0_ref[...] + t1_ref[...] + g2_ref[...])
    ls = [jnp.dot(o, wc_ref[m]) + bc_ref[m] for m in range(M)]
    out = _attention_combine(ls, ac_ref[...])
    mx = jnp.max(out, axis=1, keepdims=True)
    lse = jnp.log(jnp.sum(jnp.exp(out - mx), axis=1, keepdims=True)) + mx
    o_ref[...] = out - lse


_head = pl.pallas_call(
    _head_body,
    grid=(_GRID,),
    in_specs=[
        pl.BlockSpec((_BLK, DIM), lambda i: (i, 0)),
        pl.BlockSpec((_BLK, DIM), lambda i: (i, 0)),
        pl.BlockSpec((_BLK, DIM), lambda i: (i, 0)),
        pl.BlockSpec((_BLK, 1), lambda i: (i, 0)),
        pl.BlockSpec((M, DIM, C), lambda i: (0, 0, 0)),
        pl.BlockSpec((M, 1, C), lambda i: (0, 0, 0)),
        pl.BlockSpec((C, 1), lambda i: (0, 0)),
    ],
    out_specs=pl.BlockSpec((_BLK, C), lambda i: (i, 0)),
    out_shape=jax.ShapeDtypeStruct((N, C), jnp.float32),
)


def kernel(x, edge_index, Ws1, att1, Ws2, att2, Wc, bc, attc):
    src = edge_index[0].astype(jnp.int32)
    dst = edge_index[1].astype(jnp.int32)
    pad = E_PAD - E
    srcp1 = jnp.concatenate([src, jnp.zeros((pad,), jnp.int32)])
    dstp1 = jnp.concatenate([dst, jnp.full((pad,), N, jnp.int32)])
    srcp = srcp1.reshape(NW, CH_PER_TILE, CHUNK)
    dstp = dstp1.reshape(NW, CH_PER_TILE, CHUNK)

    degp = _deg_call()(dstp1)
    d0 = degp[0, :N, 0:1]
    d1 = degp[1, :N, 0:1]

    g1, dinv = _dense1(x, Ws1, att1.reshape(DIM, 1), d0, d1)
    sp = _prop_call()(g1, srcp, dstp)
    g2 = _dense2(sp[0, :N], sp[1, :N], g1, dinv, Ws2, att2.reshape(DIM, 1))
    tp = _prop_call()(g2, srcp, dstp)
    return _head(tp[0, :N], tp[1, :N], g2, dinv, Wc,
                 bc.reshape(M, 1, C), attc.reshape(C, 1))


# speedup vs baseline: 1.2535x; 1.2535x over previous
import functools

import jax
import jax.numpy as jnp
from jax import lax
from jax.experimental import pallas as pl
from jax.experimental.pallas import tpu as pltpu
from jax.experimental.pallas import tpu_sc as plsc

N = 10000
E = 320000
DIM = 128
C = 10
M = 3

NC = 2
NS = 16
NW = NC * NS
CHUNK = 64
NB = 4
CH_PER_TILE = 160
EPT = CH_PER_TILE * CHUNK
E_PAD = EPT * NW
ROWS_PER_TILE = 640
N_ACC = NS * ROWS_PER_TILE
DCHUNK = 128
DCH = EPT // DCHUNK


def _deg_body(dst_hbm, out_hbm, acc, idxd, vals, ssem):
    cid = lax.axis_index("c")
    sid = lax.axis_index("s")
    wid = sid * NC + cid

    zero16 = jnp.zeros((16,), jnp.float32)

    def _zrow(i, carry):
        vals[i] = zero16
        return carry

    lax.fori_loop(0, DCHUNK, _zrow, 0)
    base_row = sid * ROWS_PER_TILE
    for t in range(ROWS_PER_TILE // DCHUNK):
        pltpu.sync_copy(vals, acc.at[pl.ds(base_row + t * DCHUNK, DCHUNK)])

    one16 = jnp.ones((16,), jnp.float32)

    def _orow(i, carry):
        vals[i] = one16
        return carry

    lax.fori_loop(0, DCHUNK, _orow, 0)
    plsc.subcore_barrier()
    ebase = wid * EPT

    def _chunk(c, carry):
        off = ebase + c * DCHUNK
        pltpu.sync_copy(dst_hbm.at[pl.ds(off, DCHUNK)], idxd)
        pltpu.sync_copy(vals, acc.at[idxd], add=True)
        return carry

    lax.fori_loop(0, DCH, _chunk, 0)
    plsc.subcore_barrier()
    pltpu.sync_copy(acc.at[pl.ds(base_row, ROWS_PER_TILE)],
                    out_hbm.at[cid, pl.ds(base_row, ROWS_PER_TILE)])


@functools.cache
def _deg_call():
    return pl.kernel(
        _deg_body,
        out_type=jax.ShapeDtypeStruct((NC, N_ACC, 16), jnp.float32),
        mesh=plsc.VectorSubcoreMesh(core_axis_name="c", subcore_axis_name="s"),
        scratch_types=[
            pltpu.VMEM_SHARED((N_ACC, 16), jnp.float32),
            pltpu.VMEM((DCHUNK,), jnp.int32),
            pltpu.VMEM((DCHUNK, 16), jnp.float32),
            pltpu.SemaphoreType.DMA,
        ],
    )


FAST_CID = 0
CH_FAST = 240
CH_SLOW = 80
CH_PAIR = CH_FAST + CH_SLOW


def _prop_body(g_hbm, src_hbm, dst_hbm, out_hbm, acc,
               is0, is1, is2, is3, id0, id1, id2, id3,
               r0, r1, r2, r3, ie0, ie1, ie2, ie3,
               gs0, gs1, gs2, gs3, ss0, ss1, ss2, ss3):
    isrc = [is0, is1, is2, is3]
    idst = [id0, id1, id2, id3]
    rows = [r0, r1, r2, r3]
    isem = [ie0, ie1, ie2, ie3]
    gsem = [gs0, gs1, gs2, gs3]
    ssem = [ss0, ss1, ss2, ss3]
    cid = lax.axis_index("c")
    sid = lax.axis_index("s")

    zero16 = jnp.zeros((16,), jnp.float32)

    def _zrow(i, carry):
        r0[i // 8, pl.ds((i % 8) * 16, 16)] = zero16
        return carry

    lax.fori_loop(0, CHUNK * 8, _zrow, 0)
    base_row = sid * ROWS_PER_TILE
    for t in range(ROWS_PER_TILE // CHUNK):
        pltpu.sync_copy(r0, acc.at[pl.ds(base_row + t * CHUNK, CHUNK)])
    plsc.subcore_barrier()

    def _i_start(c, b):
        pltpu.async_copy(src_hbm.at[sid, c], isrc[b], isem[b])
        pltpu.async_copy(dst_hbm.at[sid, c], idst[b], isem[b])

    def _i_wait(c, b):
        pltpu.make_async_copy(src_hbm.at[sid, c], isrc[b], isem[b]).wait()
        pltpu.make_async_copy(dst_hbm.at[sid, c], idst[b], isem[b]).wait()

    def _g_start(b):
        pltpu.async_copy(g_hbm.at[isrc[b]], rows[b], gsem[b])

    def _g_wait(b):
        pltpu.make_async_copy(g_hbm.at[isrc[b]], rows[b], gsem[b]).wait()

    def _s_start(b):
        pltpu.async_copy(rows[b], acc.at[idst[b]], ssem[b], add=True)

    def _s_wait(b):
        pltpu.make_async_copy(rows[b], acc.at[idst[b]], ssem[b]).wait()

    def _run(ch, choff):
        for c in range(3):
            _i_start(choff + c, c)
        for c in range(2):
            _i_wait(choff + c, c)
            _g_start(c)

        def _step(i, k, sw_ok, is_ok, gs_ok):
            if sw_ok:
                _s_wait((k + 3) % NB)
            if is_ok:
                _i_start(i + 3, (k + 3) % NB)
            if gs_ok:
                _i_wait(i + 2, (k + 2) % NB)
                _g_start((k + 2) % NB)
            _g_wait(k)
            _s_start(k)

        for k in range(NB):
            _step(choff + k, k, k >= 1, True, True)

        def _group(g, carry):
            i0 = choff + g * NB
            for k in range(NB):
                _step(i0 + k, k, True, True, True)
            return carry

        lax.fori_loop(1, ch // NB - 1, _group, 0)
        ilast = ch - NB
        for k in range(NB):
            i = ilast + k
            _step(choff + i, k, True, i + 3 < ch, i + 2 < ch)
        _s_wait((ch - 1) % NB)

    @pl.when(cid == FAST_CID)
    def _fast():
        _run(CH_FAST, 0 if FAST_CID == 0 else CH_SLOW)

    @pl.when(cid != FAST_CID)
    def _slow():
        _run(CH_SLOW, CH_FAST if FAST_CID == 0 else 0)

    plsc.subcore_barrier()
    pltpu.sync_copy(acc.at[pl.ds(base_row, ROWS_PER_TILE)],
                    out_hbm.at[cid, pl.ds(base_row, ROWS_PER_TILE)])


@functools.cache
def _prop_call():
    return pl.kernel(
        _prop_body,
        out_type=jax.ShapeDtypeStruct((NC, N_ACC, DIM), jnp.float32),
        mesh=plsc.VectorSubcoreMesh(core_axis_name="c", subcore_axis_name="s"),
        scratch_types=(
            [pltpu.VMEM_SHARED((N_ACC, DIM), jnp.float32)]
            + [pltpu.VMEM((CHUNK,), jnp.int32)] * 8
            + [pltpu.VMEM((CHUNK, DIM), jnp.float32)] * 4
            + [pltpu.SemaphoreType.DMA] * 12
        ),
    )


_BLK = 1000
_GRID = N // _BLK


def _attention_combine(h_list, att):
    ss = [jnp.dot(jnp.tanh(h), att) for h in h_list]
    mx = jnp.maximum(jnp.maximum(ss[0], ss[1]), ss[2])
    es = [jnp.exp(s - mx) for s in ss]
    z = es[0] + es[1] + es[2]
    return (es[0] * h_list[0] + es[1] * h_list[1] + es[2] * h_list[2]) / z


def _dense1_body(x_ref, w_ref, a_ref, d0_ref, d1_ref, g_ref, dinv_ref):
    dinv = lax.rsqrt(d0_ref[...] + d1_ref[...] + 1.0)
    x = x_ref[...]
    hs = [jnp.dot(x, w_ref[m]) for m in range(M)]
    hagg = _attention_combine(hs, a_ref[...])
    dinv_ref[...] = dinv
    g_ref[...] = hagg * dinv


_dense1 = pl.pallas_call(
    _dense1_body,
    grid=(_GRID,),
    in_specs=[
        pl.BlockSpec((_BLK, DIM), lambda i: (i, 0)),
        pl.BlockSpec((M, DIM, DIM), lambda i: (0, 0, 0)),
        pl.BlockSpec((DIM, 1), lambda i: (0, 0)),
        pl.BlockSpec((_BLK, 1), lambda i: (i, 0)),
        pl.BlockSpec((_BLK, 1), lambda i: (i, 0)),
    ],
    out_specs=[
        pl.BlockSpec((_BLK, DIM), lambda i: (i, 0)),
        pl.BlockSpec((_BLK, 1), lambda i: (i, 0)),
    ],
    out_shape=[
        jax.ShapeDtypeStruct((N, DIM), jnp.float32),
        jax.ShapeDtypeStruct((N, 1), jnp.float32),
    ],
)


def _dense2_body(s0_ref, s1_ref, g1_ref, dinv_ref, w_ref, a_ref, g2_ref):
    dinv = dinv_ref[...]
    h = jnp.maximum(dinv * (s0_ref[...] + s1_ref[...] + g1_ref[...]), 0.0)
    hs = [jnp.dot(h, w_ref[m]) for m in range(M)]
    hagg = _attention_combine(hs, a_ref[...])
    g2_ref[...] = hagg * dinv


_dense2 = pl.pallas_call(
    _dense2_body,
    grid=(_GRID,),
    in_specs=[
        pl.BlockSpec((_BLK, DIM), lambda i: (i, 0)),
        pl.BlockSpec((_BLK, DIM), lambda i: (i, 0)),
        pl.BlockSpec((_BLK, DIM), lambda i: (i, 0)),
        pl.BlockSpec((_BLK, 1), lambda i: (i, 0)),
        pl.BlockSpec((M, DIM, DIM), lambda i: (0, 0, 0)),
        pl.BlockSpec((DIM, 1), lambda i: (0, 0)),
    ],
    out_specs=pl.BlockSpec((_BLK, DIM), lambda i: (i, 0)),
    out_shape=jax.ShapeDtypeStruct((N, DIM), jnp.float32),
)


def _head_body(t0_ref, t1_ref, g2_ref, dinv_ref, wc_ref, bc_ref, ac_ref, o_ref):
    o = dinv_ref[...] * (t0_ref[...] + t1_ref[...] + g2_ref[...])
    ls = [jnp.dot(o, wc_ref[m]) + bc_ref[m] for m in range(M)]
    out = _attention_combine(ls, ac_ref[...])
    mx = jnp.max(out, axis=1, keepdims=True)
    lse = jnp.log(jnp.sum(jnp.exp(out - mx), axis=1, keepdims=True)) + mx
    o_ref[...] = out - lse


_head = pl.pallas_call(
    _head_body,
    grid=(_GRID,),
    in_specs=[
        pl.BlockSpec((_BLK, DIM), lambda i: (i, 0)),
        pl.BlockSpec((_BLK, DIM), lambda i: (i, 0)),
        pl.BlockSpec((_BLK, DIM), lambda i: (i, 0)),
        pl.BlockSpec((_BLK, 1), lambda i: (i, 0)),
        pl.BlockSpec((M, DIM, C), lambda i: (0, 0, 0)),
        pl.BlockSpec((M, 1, C), lambda i: (0, 0, 0)),
        pl.BlockSpec((C, 1), lambda i: (0, 0)),
    ],
    out_specs=pl.BlockSpec((_BLK, C), lambda i: (i, 0)),
    out_shape=jax.ShapeDtypeStruct((N, C), jnp.float32),
)


def kernel(x, edge_index, Ws1, att1, Ws2, att2, Wc, bc, attc):
    src = edge_index[0].astype(jnp.int32)
    dst = edge_index[1].astype(jnp.int32)
    pad = E_PAD - E
    srcp1 = jnp.concatenate([src, jnp.zeros((pad,), jnp.int32)])
    dstp1 = jnp.concatenate([dst, jnp.full((pad,), N, jnp.int32)])
    srcp = srcp1.reshape(NS, CH_PAIR, CHUNK)
    dstp = dstp1.reshape(NS, CH_PAIR, CHUNK)

    degp = _deg_call()(dstp1)
    d0 = degp[0, :N, 0:1]
    d1 = degp[1, :N, 0:1]

    g1, dinv = _dense1(x, Ws1, att1.reshape(DIM, 1), d0, d1)
    sp = _prop_call()(g1, srcp, dstp)
    g2 = _dense2(sp[0, :N], sp[1, :N], g1, dinv, Ws2, att2.reshape(DIM, 1))
    tp = _prop_call()(g2, srcp, dstp)
    return _head(tp[0, :N], tp[1, :N], g2, dinv, Wc,
                 bc.reshape(M, 1, C), attc.reshape(C, 1))


# speedup vs baseline: 1.2594x; 1.0047x over previous
import functools

import jax
import jax.numpy as jnp
from jax import lax
from jax.experimental import pallas as pl
from jax.experimental.pallas import tpu as pltpu
from jax.experimental.pallas import tpu_sc as plsc

N = 10000
E = 320000
DIM = 128
C = 10
M = 3

NC = 2
NS = 16
NW = NC * NS
CHUNK = 64
NB = 4
CH_PER_TILE = 160
EPT = CH_PER_TILE * CHUNK
E_PAD = EPT * NW
ROWS_PER_TILE = 640
N_ACC = NS * ROWS_PER_TILE
DCHUNK = 128
DCH = EPT // DCHUNK


def _deg_body(dst_hbm, out_hbm, acc, idxd, vals, ssem):
    cid = lax.axis_index("c")
    sid = lax.axis_index("s")
    wid = sid * NC + cid

    zero16 = jnp.zeros((16,), jnp.float32)

    def _zrow(i, carry):
        vals[i] = zero16
        return carry

    lax.fori_loop(0, DCHUNK, _zrow, 0)
    base_row = sid * ROWS_PER_TILE
    for t in range(ROWS_PER_TILE // DCHUNK):
        pltpu.sync_copy(vals, acc.at[pl.ds(base_row + t * DCHUNK, DCHUNK)])

    one16 = jnp.ones((16,), jnp.float32)

    def _orow(i, carry):
        vals[i] = one16
        return carry

    lax.fori_loop(0, DCHUNK, _orow, 0)
    plsc.subcore_barrier()
    ebase = wid * EPT

    def _chunk(c, carry):
        off = ebase + c * DCHUNK
        pltpu.sync_copy(dst_hbm.at[pl.ds(off, DCHUNK)], idxd)
        pltpu.sync_copy(vals, acc.at[idxd], add=True)
        return carry

    lax.fori_loop(0, DCH, _chunk, 0)
    plsc.subcore_barrier()
    pltpu.sync_copy(acc.at[pl.ds(base_row, ROWS_PER_TILE)],
                    out_hbm.at[cid, pl.ds(base_row, ROWS_PER_TILE)])


@functools.cache
def _deg_call():
    return pl.kernel(
        _deg_body,
        out_type=jax.ShapeDtypeStruct((NC, N_ACC, 16), jnp.float32),
        mesh=plsc.VectorSubcoreMesh(core_axis_name="c", subcore_axis_name="s"),
        scratch_types=[
            pltpu.VMEM_SHARED((N_ACC, 16), jnp.float32),
            pltpu.VMEM((DCHUNK,), jnp.int32),
            pltpu.VMEM((DCHUNK, 16), jnp.float32),
            pltpu.SemaphoreType.DMA,
        ],
    )


FAST_CID = 0
CH_FAST = 240
CH_SLOW = 80
CH_PAIR = CH_FAST + CH_SLOW


def _prop_body(g_hbm, src_hbm, dst_hbm, out_hbm, acc,
               is0, is1, is2, is3, id0, id1, id2, id3,
               r0, r1, r2, r3, ie0, ie1, ie2, ie3,
               gs0, gs1, gs2, gs3, ss0, ss1, ss2, ss3):
    isrc = [is0, is1, is2, is3]
    idst = [id0, id1, id2, id3]
    rows = [r0, r1, r2, r3]
    isem = [ie0, ie1, ie2, ie3]
    gsem = [gs0, gs1, gs2, gs3]
    ssem = [ss0, ss1, ss2, ss3]
    cid = lax.axis_index("c")
    sid = lax.axis_index("s")

    zero16 = jnp.zeros((16,), jnp.float32)

    def _zrow(i, carry):
        r0[i // 8, pl.ds((i % 8) * 16, 16)] = zero16
        return carry

    lax.fori_loop(0, CHUNK * 8, _zrow, 0)
    base_row = sid * ROWS_PER_TILE
    for t in range(ROWS_PER_TILE // CHUNK):
        pltpu.sync_copy(r0, acc.at[pl.ds(base_row + t * CHUNK, CHUNK)])
    plsc.subcore_barrier()

    def _i_start(c, b):
        pltpu.async_copy(src_hbm.at[sid, c], isrc[b], isem[b])
        pltpu.async_copy(dst_hbm.at[sid, c], idst[b], isem[b])

    def _i_wait(c, b):
        pltpu.make_async_copy(src_hbm.at[sid, c], isrc[b], isem[b]).wait()
        pltpu.make_async_copy(dst_hbm.at[sid, c], idst[b], isem[b]).wait()

    def _g_start(b):
        pltpu.async_copy(g_hbm.at[isrc[b]], rows[b], gsem[b])

    def _g_wait(b):
        pltpu.make_async_copy(g_hbm.at[isrc[b]], rows[b], gsem[b]).wait()

    def _s_start(b):
        pltpu.async_copy(rows[b], acc.at[idst[b]], ssem[b], add=True)

    def _s_wait(b):
        pltpu.make_async_copy(rows[b], acc.at[idst[b]], ssem[b]).wait()

    def _run(ch, choff):
        for c in range(3):
            _i_start(choff + c, c)
        for c in range(2):
            _i_wait(choff + c, c)
            _g_start(c)

        def _step(i, k, sw_ok, is_ok, gs_ok):
            if sw_ok:
                _s_wait((k + 3) % NB)
            if is_ok:
                _i_start(i + 3, (k + 3) % NB)
            if gs_ok:
                _i_wait(i + 2, (k + 2) % NB)
                _g_start((k + 2) % NB)
            _g_wait(k)
            _s_start(k)

        for k in range(NB):
            _step(choff + k, k, k >= 1, True, True)

        def _group(g, carry):
            i0 = choff + g * NB
            for k in range(NB):
                _step(i0 + k, k, True, True, True)
            return carry

        lax.fori_loop(1, ch // NB - 1, _group, 0)
        ilast = ch - NB
        for k in range(NB):
            i = ilast + k
            _step(choff + i, k, True, i + 3 < ch, i + 2 < ch)
        _s_wait((ch - 1) % NB)

    @pl.when(cid == FAST_CID)
    def _fast():
        _run(CH_FAST, 0 if FAST_CID == 0 else CH_SLOW)

    @pl.when(cid != FAST_CID)
    def _slow():
        _run(CH_SLOW, CH_FAST if FAST_CID == 0 else 0)

    plsc.subcore_barrier()
    pltpu.sync_copy(acc.at[pl.ds(base_row, ROWS_PER_TILE)],
                    out_hbm.at[cid, pl.ds(base_row, ROWS_PER_TILE)])


@functools.cache
def _prop_call():
    return pl.kernel(
        _prop_body,
        out_type=jax.ShapeDtypeStruct((NC, N_ACC, DIM), jnp.float32),
        mesh=plsc.VectorSubcoreMesh(core_axis_name="c", subcore_axis_name="s"),
        scratch_types=(
            [pltpu.VMEM_SHARED((N_ACC, DIM), jnp.float32)]
            + [pltpu.VMEM((CHUNK,), jnp.int32)] * 8
            + [pltpu.VMEM((CHUNK, DIM), jnp.float32)] * 4
            + [pltpu.SemaphoreType.DMA] * 12
        ),
    )


_BLK = 1000
_GRID = N // _BLK


def _attention_combine(h_list, att):
    ss = [jnp.dot(jnp.tanh(h), att) for h in h_list]
    mx = jnp.maximum(jnp.maximum(ss[0], ss[1]), ss[2])
    es = [jnp.exp(s - mx) for s in ss]
    z = es[0] + es[1] + es[2]
    return (es[0] * h_list[0] + es[1] * h_list[1] + es[2] * h_list[2]) / z


def _dense1a_body(x_ref, w_ref, a_ref, h_ref):
    x = x_ref[...]
    hs = [jnp.dot(x, w_ref[m]) for m in range(M)]
    h_ref[...] = _attention_combine(hs, a_ref[...])


_dense1a = pl.pallas_call(
    _dense1a_body,
    grid=(_GRID,),
    in_specs=[
        pl.BlockSpec((_BLK, DIM), lambda i: (i, 0)),
        pl.BlockSpec((M, DIM, DIM), lambda i: (0, 0, 0)),
        pl.BlockSpec((DIM, 1), lambda i: (0, 0)),
    ],
    out_specs=pl.BlockSpec((_BLK, DIM), lambda i: (i, 0)),
    out_shape=jax.ShapeDtypeStruct((N, DIM), jnp.float32),
)


def _dense1b_body(h_ref, d0_ref, d1_ref, g_ref, dinv_ref):
    dinv = lax.rsqrt(d0_ref[...] + d1_ref[...] + 1.0)
    dinv_ref[...] = dinv
    g_ref[...] = h_ref[...] * dinv


_dense1b = pl.pallas_call(
    _dense1b_body,
    grid=(_GRID,),
    in_specs=[
        pl.BlockSpec((_BLK, DIM), lambda i: (i, 0)),
        pl.BlockSpec((_BLK, 1), lambda i: (i, 0)),
        pl.BlockSpec((_BLK, 1), lambda i: (i, 0)),
    ],
    out_specs=[
        pl.BlockSpec((_BLK, DIM), lambda i: (i, 0)),
        pl.BlockSpec((_BLK, 1), lambda i: (i, 0)),
    ],
    out_shape=[
        jax.ShapeDtypeStruct((N, DIM), jnp.float32),
        jax.ShapeDtypeStruct((N, 1), jnp.float32),
    ],
)


def _dense2_body(s0_ref, s1_ref, g1_ref, dinv_ref, w_ref, a_ref, g2_ref):
    dinv = dinv_ref[...]
    h = jnp.maximum(dinv * (s0_ref[...] + s1_ref[...] + g1_ref[...]), 0.0)
    hs = [jnp.dot(h, w_ref[m]) for m in range(M)]
    hagg = _attention_combine(hs, a_ref[...])
    g2_ref[...] = hagg * dinv


_dense2 = pl.pallas_call(
    _dense2_body,
    grid=(_GRID,),
    in_specs=[
        pl.BlockSpec((_BLK, DIM), lambda i: (i, 0)),
        pl.BlockSpec((_BLK, DIM), lambda i: (i, 0)),
        pl.BlockSpec((_BLK, DIM), lambda i: (i, 0)),
        pl.BlockSpec((_BLK, 1), lambda i: (i, 0)),
        pl.BlockSpec((M, DIM, DIM), lambda i: (0, 0, 0)),
        pl.BlockSpec((DIM, 1), lambda i: (0, 0)),
    ],
    out_specs=pl.BlockSpec((_BLK, DIM), lambda i: (i, 0)),
    out_shape=jax.ShapeDtypeStruct((N, DIM), jnp.float32),
)


def _head_body(t0_ref, t1_ref, g2_ref, dinv_ref, wc_ref, bc_ref, ac_ref, o_ref):
    o = dinv_ref[...] * (t0_ref[...] + t1_ref[...] + g2_ref[...])
    ls = [jnp.dot(o, wc_ref[m]) + bc_ref[m] for m in range(M)]
    out = _attention_combine(ls, ac_ref[...])
    mx = jnp.max(out, axis=1, keepdims=True)
    lse = jnp.log(jnp.sum(jnp.exp(out - mx), axis=1, keepdims=True)) + mx
    o_ref[...] = out - lse


_head = pl.pallas_call(
    _head_body,
    grid=(_GRID,),
    in_specs=[
        pl.BlockSpec((_BLK, DIM), lambda i: (i, 0)),
        pl.BlockSpec((_BLK, DIM), lambda i: (i, 0)),
        pl.BlockSpec((_BLK, DIM), lambda i: (i, 0)),
        pl.BlockSpec((_BLK, 1), lambda i: (i, 0)),
        pl.BlockSpec((M, DIM, C), lambda i: (0, 0, 0)),
        pl.BlockSpec((M, 1, C), lambda i: (0, 0, 0)),
        pl.BlockSpec((C, 1), lambda i: (0, 0)),
    ],
    out_specs=pl.BlockSpec((_BLK, C), lambda i: (i, 0)),
    out_shape=jax.ShapeDtypeStruct((N, C), jnp.float32),
)


def kernel(x, edge_index, Ws1, att1, Ws2, att2, Wc, bc, attc):
    src = edge_index[0].astype(jnp.int32)
    dst = edge_index[1].astype(jnp.int32)
    pad = E_PAD - E
    srcp1 = jnp.concatenate([src, jnp.zeros((pad,), jnp.int32)])
    dstp1 = jnp.concatenate([dst, jnp.full((pad,), N, jnp.int32)])
    srcp = srcp1.reshape(NS, CH_PAIR, CHUNK)
    dstp = dstp1.reshape(NS, CH_PAIR, CHUNK)

    degp = _deg_call()(dstp1)
    d0 = degp[0, :N, 0:1]
    d1 = degp[1, :N, 0:1]

    hagg1 = _dense1a(x, Ws1, att1.reshape(DIM, 1))
    g1, dinv = _dense1b(hagg1, d0, d1)
    sp = _prop_call()(g1, srcp, dstp)
    g2 = _dense2(sp[0, :N], sp[1, :N], g1, dinv, Ws2, att2.reshape(DIM, 1))
    tp = _prop_call()(g2, srcp, dstp)
    return _head(tp[0, :N], tp[1, :N], g2, dinv, Wc,
                 bc.reshape(M, 1, C), attc.reshape(C, 1))
